# baseline (speedup 1.0000x reference)
import functools

import jax
import jax.numpy as jnp
from jax import lax
from jax.experimental import pallas as pl
from jax.experimental.pallas import tpu as pltpu
from jax.experimental.pallas import tpu_sc as plsc

B, S, D = 1, 2048, 1024
H, KVH, DH = 16, 4, 64
E, K, F = 8, 2, 1024
EPS = 1e-5
ALPHA = 1.702
LIMIT = 7.0
SCALE = DH ** -0.5

BS = 256
NB = S // BS
BQ = 256
BK = 256
NQ = S // BQ
NK = S // BK
BT = 256
NSLOT = S * K + E * BT
NBLK = NSLOT // BT
QD = H * DH
KD = KVH * DH

SC_NC = 2
SC_NS = 16
SC_NW = SC_NC * SC_NS
SC_CHUNK = 64


def _qkv_body(x_ref, w1_ref, w_ref, cq_ref, sq_ref, ck_ref, sk_ref,
              tq_ref, tk_ref, vb_ref, q_ref, k_ref, v_ref):
    x = x_ref[...]
    rs = lax.rsqrt(jnp.mean(x * x, axis=1, keepdims=True) + EPS)
    xn = x * rs * w1_ref[...]
    qkv = jnp.dot(xn, w_ref[...], preferred_element_type=jnp.float32,
                  precision=lax.Precision.HIGHEST)
    qa = qkv[:, :QD]
    qb = qkv[:, QD:2 * QD]
    ka = qkv[:, 2 * QD:2 * QD + KD]
    kb = qkv[:, 2 * QD + KD:2 * QD + 2 * KD]
    v = qkv[:, 2 * QD + 2 * KD:]
    q = qa * cq_ref[...] + qb * sq_ref[...] + tq_ref[...]
    k = ka * ck_ref[...] + kb * sk_ref[...] + tk_ref[...]
    v = v + vb_ref[...]
    q_ref[...] = q.reshape(BS, H, DH).transpose(1, 0, 2)
    k_ref[...] = k.reshape(BS, KVH, DH).transpose(1, 0, 2)
    v_ref[...] = v.reshape(BS, KVH, DH).transpose(1, 0, 2)


def _run_qkv(x2d, norm1_w, wqkv, cq, sq, ck, sk, tq, tk, v_b):
    full = lambda a: pl.BlockSpec(a.shape, lambda i: (0,) * a.ndim)
    blk = lambda w: pl.BlockSpec((BS, w), lambda i: (i, 0))
    return pl.pallas_call(
        _qkv_body,
        grid=(NB,),
        in_specs=[blk(D), full(norm1_w), full(wqkv),
                  blk(QD), blk(QD), blk(KD), blk(KD),
                  blk(QD), blk(KD), full(v_b)],
        out_specs=[pl.BlockSpec((H, BS, DH), lambda i: (0, i, 0)),
                   pl.BlockSpec((KVH, BS, DH), lambda i: (0, i, 0)),
                   pl.BlockSpec((KVH, BS, DH), lambda i: (0, i, 0))],
        out_shape=[jax.ShapeDtypeStruct((H, S, DH), jnp.float32),
                   jax.ShapeDtypeStruct((KVH, S, DH), jnp.float32),
                   jax.ShapeDtypeStruct((KVH, S, DH), jnp.float32)],
    )(x2d, norm1_w, wqkv, cq, sq, ck, sk, tq, tk, v_b)


def _flash_body(q_ref, k_ref, v_ref, sink_ref, o_ref, m_ref, l_ref, acc_ref):
    h = pl.program_id(0)
    qb = pl.program_id(1)
    kb = pl.program_id(2)

    @pl.when(kb == 0)
    def _init():
        m_ref[...] = jnp.full_like(m_ref, -1e30)
        l_ref[...] = jnp.zeros_like(l_ref)
        acc_ref[...] = jnp.zeros_like(acc_ref)

    @pl.when(kb <= qb)
    def _step():
        q = q_ref[0]
        k = k_ref[0]
        s = lax.dot_general(q, k, (((1,), (1,)), ((), ())),
                            preferred_element_type=jnp.float32,
                            precision=lax.Precision.HIGHEST) * SCALE
        rows = qb * BQ + lax.broadcasted_iota(jnp.int32, (BQ, BK), 0)
        cols = kb * BK + lax.broadcasted_iota(jnp.int32, (BQ, BK), 1)
        s = jnp.where((kb < qb) | (cols <= rows), s, -1e30)
        m_old = m_ref[...]
        m_new = jnp.maximum(m_old, jnp.max(s, axis=1, keepdims=True))
        p = jnp.exp(s - m_new)
        alpha = jnp.exp(m_old - m_new)
        m_ref[...] = m_new
        l_ref[...] = l_ref[...] * alpha + jnp.sum(p, axis=1, keepdims=True)
        acc_ref[...] = acc_ref[...] * alpha + lax.dot_general(
            p, v_ref[0], (((1,), (0,)), ((), ())),
            preferred_element_type=jnp.float32,
            precision=lax.Precision.HIGHEST)

    @pl.when(kb == NK - 1)
    def _fin():
        sink = sink_ref[0, h]
        m_old = m_ref[...]
        m_fin = jnp.maximum(m_old, sink)
        corr = jnp.exp(m_old - m_fin)
        l_fin = l_ref[...] * corr + jnp.exp(sink - m_fin)
        o_ref[0] = acc_ref[...] * corr / l_fin


def _run_flash(q, k, v, sinks2d):
    return pl.pallas_call(
        _flash_body,
        grid=(H, NQ, NK),
        in_specs=[
            pl.BlockSpec((1, BQ, DH), lambda h, qb, kb: (h, qb, 0)),
            pl.BlockSpec((1, BK, DH),
                         lambda h, qb, kb: (h // (H // KVH), kb, 0)),
            pl.BlockSpec((1, BK, DH),
                         lambda h, qb, kb: (h // (H // KVH), kb, 0)),
            pl.BlockSpec(memory_space=pltpu.SMEM),
        ],
        out_specs=pl.BlockSpec((1, BQ, DH), lambda h, qb, kb: (h, qb, 0)),
        out_shape=jax.ShapeDtypeStruct((H, S, DH), jnp.float32),
        scratch_shapes=[pltpu.VMEM((BQ, 1), jnp.float32),
                        pltpu.VMEM((BQ, 1), jnp.float32),
                        pltpu.VMEM((BQ, DH), jnp.float32)],
    )(q, k, v, sinks2d)


def _oproj_body(a_ref, wo_ref, ob_ref, x_ref, w2_ref, wg_ref, gb_ref,
                h_ref, hn_ref, lg_ref):
    attn = a_ref[...].transpose(1, 0, 2).reshape(BS, QD)
    a = jnp.dot(attn, wo_ref[...], preferred_element_type=jnp.float32,
                precision=lax.Precision.HIGHEST)
    hid = a + ob_ref[...] + x_ref[...]
    h_ref[...] = hid
    rs = lax.rsqrt(jnp.mean(hid * hid, axis=1, keepdims=True) + EPS)
    hn = hid * rs * w2_ref[...]
    hn_ref[...] = hn
    lg_ref[...] = jnp.dot(hn, wg_ref[...],
                          preferred_element_type=jnp.float32,
                          precision=lax.Precision.HIGHEST) + gb_ref[...]


def _run_oproj(attn, wo, o_b, x2d, norm2_w, wg_pad, gb_pad):
    full = lambda a: pl.BlockSpec(a.shape, lambda i: (0,) * a.ndim)
    blk = lambda w: pl.BlockSpec((BS, w), lambda i: (i, 0))
    return pl.pallas_call(
        _oproj_body,
        grid=(NB,),
        in_specs=[pl.BlockSpec((H, BS, DH), lambda i: (0, i, 0)),
                  full(wo), full(o_b), blk(D), full(norm2_w),
                  full(wg_pad), full(gb_pad)],
        out_specs=[blk(D), blk(D), blk(128)],
        out_shape=[jax.ShapeDtypeStruct((S, D), jnp.float32),
                   jax.ShapeDtypeStruct((S, D), jnp.float32),
                   jax.ShapeDtypeStruct((S, 128), jnp.float32)],
    )(attn, wo, o_b, x2d, norm2_w, wg_pad, gb_pad)


def _make_sc_gather(n_rows, d, dtype):
    rows_pw = n_rows // SC_NW
    n_chunks = rows_pw // SC_CHUNK
    assert rows_pw % SC_CHUNK == 0
    mesh = plsc.VectorSubcoreMesh(core_axis_name="c", subcore_axis_name="s")

    @functools.partial(
        pl.kernel, mesh=mesh,
        out_type=jax.ShapeDtypeStruct((n_rows, d), dtype),
        scratch_types=[pltpu.VMEM((SC_CHUNK,), jnp.int32),
                       pltpu.VMEM((SC_CHUNK, d), dtype),
                       pltpu.SemaphoreType.DMA],
    )
    def gather(src_hbm, idx_hbm, out_hbm, idx_v, rows_v, sem):
        wid = lax.axis_index("s") * SC_NC + lax.axis_index("c")
        for c in range(n_chunks):
            base = wid * rows_pw + c * SC_CHUNK
            pltpu.sync_copy(idx_hbm.at[pl.ds(base, SC_CHUNK)], idx_v)
            pltpu.async_copy(src_hbm.at[idx_v], rows_v, sem).wait()
            pltpu.sync_copy(rows_v, out_hbm.at[pl.ds(base, SC_CHUNK)])

    return gather


def _ffn_body(be_ref, x_ref, gp_ref, gpb_ref, up_ref, upb_ref,
              dp_ref, dpb_ref, sw_ref, y_ref):
    x = x_ref[...].astype(jnp.bfloat16)
    g = jnp.dot(x, gp_ref[0], preferred_element_type=jnp.float32) + gpb_ref[0]
    u = jnp.dot(x, up_ref[0], preferred_element_type=jnp.float32) + upb_ref[0]
    g = jnp.minimum(g, LIMIT)
    u = jnp.clip(u, -LIMIT, LIMIT)
    glu = g / (1.0 + jnp.exp(-ALPHA * g))
    act = ((u + 1.0) * glu).astype(jnp.bfloat16)
    dn = jnp.dot(act, dp_ref[0], preferred_element_type=jnp.float32) + dpb_ref[0]
    y_ref[...] = dn * sw_ref[0, 0, :][:, None]


def _run_ffn(x_sorted, block_expert, gp_w, gp_b, up_w, up_b, dp_w, dp_b, sw3):
    grid_spec = pltpu.PrefetchScalarGridSpec(
        num_scalar_prefetch=1,
        grid=(NBLK,),
        in_specs=[
            pl.BlockSpec((BT, D), lambda b, be: (b, 0)),
            pl.BlockSpec((1, D, F), lambda b, be: (be[b], 0, 0)),
            pl.BlockSpec((1, 1, F), lambda b, be: (be[b], 0, 0)),
            pl.BlockSpec((1, D, F), lambda b, be: (be[b], 0, 0)),
            pl.BlockSpec((1, 1, F), lambda b, be: (be[b], 0, 0)),
            pl.BlockSpec((1, F, D), lambda b, be: (be[b], 0, 0)),
            pl.BlockSpec((1, 1, D), lambda b, be: (be[b], 0, 0)),
            pl.BlockSpec((1, 1, BT), lambda b, be: (b, 0, 0)),
        ],
        out_specs=pl.BlockSpec((BT, D), lambda b, be: (b, 0)),
    )
    return pl.pallas_call(
        _ffn_body,
        grid_spec=grid_spec,
        out_shape=jax.ShapeDtypeStruct((NSLOT, D), jnp.float32),
    )(block_expert, x_sorted, gp_w, gp_b, up_w, up_b, dp_w, dp_b, sw3)


def _combine_body(h_ref, y1_ref, y2_ref, o_ref):
    o_ref[...] = h_ref[...] + y1_ref[...] + y2_ref[...]


def _run_combine(h2d, yg):
    blk = pl.BlockSpec((BS, D), lambda i: (i, 0))
    return pl.pallas_call(
        _combine_body,
        grid=(NB,),
        in_specs=[blk,
                  pl.BlockSpec((BS, D), lambda i: (i, 0)),
                  pl.BlockSpec((BS, D), lambda i: (i + NB, 0))],
        out_specs=blk,
        out_shape=jax.ShapeDtypeStruct((S, D), jnp.float32),
    )(h2d, yg, yg)


def _rot_cols(w):
    nh = w.shape[1] // DH
    w3 = w.reshape(w.shape[0], nh, DH)
    return jnp.concatenate([-w3[:, :, DH // 2:], w3[:, :, :DH // 2]],
                           axis=2).reshape(w.shape)


def _rot_vec(b):
    nh = b.shape[0] // DH
    b2 = b.reshape(nh, DH)
    return jnp.concatenate([-b2[:, DH // 2:], b2[:, :DH // 2]],
                           axis=1).reshape(-1)


def kernel(hidden_states, cos, sin, norm1_w, norm2_w, q_w, q_b, k_w, k_b,
           v_w, v_b, o_w, o_b, sinks, gate_w, gate_b, gp_w, gp_b, up_w,
           up_b, dp_w, dp_b):
    x2d = hidden_states.reshape(S, D)
    cos2 = cos.reshape(S, DH)
    sin2 = sin.reshape(S, DH)

    wq = q_w.T
    wk = k_w.T
    wv = v_w.T
    wqkv = jnp.concatenate(
        [wq, _rot_cols(wq), wk, _rot_cols(wk), wv], axis=1)
    cq = jnp.tile(cos2, (1, H))
    sq = jnp.tile(sin2, (1, H))
    ck = jnp.tile(cos2, (1, KVH))
    sk = jnp.tile(sin2, (1, KVH))
    tq = cq * q_b[None, :] + sq * _rot_vec(q_b)[None, :]
    tk = ck * k_b[None, :] + sk * _rot_vec(k_b)[None, :]

    q, k, v = _run_qkv(x2d, norm1_w.reshape(1, D), wqkv, cq, sq, ck, sk,
                       tq, tk, v_b.reshape(1, KD))

    attn = _run_flash(q, k, v, sinks.reshape(1, H))

    wg_pad = jnp.zeros((D, 128), jnp.float32).at[:, :E].set(gate_w.T)
    gb_pad = jnp.zeros((1, 128), jnp.float32).at[0, :E].set(gate_b)
    h2d, hn2d, lg_pad = _run_oproj(
        attn, o_w.T, o_b.reshape(1, D), x2d,
        norm2_w.reshape(1, D), wg_pad, gb_pad)

    logits = lg_pad[:, :E]
    tv, ti = lax.top_k(logits, K)
    tw = jax.nn.softmax(tv, axis=-1)
    expert_ids = ti.reshape(-1)
    pair_w = tw.reshape(-1)
    order = jnp.argsort(expert_ids)
    sorted_e = expert_ids[order]
    counts = jnp.sum(expert_ids[None, :] == jnp.arange(E)[:, None], axis=1)
    padded = ((counts + BT - 1) // BT) * BT
    pad_off = jnp.concatenate([jnp.zeros(1, jnp.int32),
                               jnp.cumsum(padded).astype(jnp.int32)])
    raw_off = jnp.concatenate([jnp.zeros(1, jnp.int32),
                               jnp.cumsum(counts).astype(jnp.int32)])
    slot_pos = pad_off[sorted_e] + (
        jnp.arange(S * K, dtype=jnp.int32) - raw_off[sorted_e])
    slot_token = jnp.zeros(NSLOT, jnp.int32).at[slot_pos].set(
        (order // K).astype(jnp.int32))
    slot_w = jnp.zeros(NSLOT, jnp.float32).at[slot_pos].set(pair_w[order])
    inv_slot = jnp.zeros(S * K, jnp.int32).at[order].set(slot_pos)
    block_expert = jnp.clip(
        jnp.searchsorted(pad_off, jnp.arange(NBLK, dtype=jnp.int32) * BT,
                         side='right') - 1, 0, E - 1).astype(jnp.int32)

    x_sorted = _make_sc_gather(NSLOT, D, jnp.float32)(hn2d, slot_token)

    y = _run_ffn(x_sorted, block_expert,
                 gp_w.astype(jnp.bfloat16), gp_b.reshape(E, 1, F),
                 up_w.astype(jnp.bfloat16), up_b.reshape(E, 1, F),
                 dp_w.astype(jnp.bfloat16), dp_b.reshape(E, 1, D),
                 slot_w.reshape(NBLK, 1, BT))

    inv2 = inv_slot.reshape(S, K)
    gidx = jnp.concatenate([inv2[:, 0], inv2[:, 1]])
    yg = _make_sc_gather(2 * S, D, jnp.float32)(y, gidx)

    out = _run_combine(h2d, yg)
    return out.reshape(B, S, D)


# speedup vs baseline: 1.4555x; 1.4555x over previous
import functools

import jax
import jax.numpy as jnp
from jax import lax
from jax.experimental import pallas as pl
from jax.experimental.pallas import tpu as pltpu
from jax.experimental.pallas import tpu_sc as plsc

B, S, D = 1, 2048, 1024
H, KVH, DH = 16, 4, 64
E, K, F = 8, 2, 1024
EPS = 1e-5
ALPHA = 1.702
LIMIT = 7.0
SCALE = DH ** -0.5
_PREC = lax.Precision.HIGHEST

BS = 256
NB = S // BS
BQ = 256
BK = 256
NQ = S // BQ
NK = S // BK
BT = 256
NSLOT = S * K + E * BT
NBLK = NSLOT // BT
QD = H * DH
KD = KVH * DH

SC_NC = 2
SC_NS = 16
SC_NW = SC_NC * SC_NS
SC_CHUNK = 64


def _qkv_body(x_ref, w1_ref, w_ref, cq_ref, sq_ref, ck_ref, sk_ref,
              tq_ref, tk_ref, vb_ref, q_ref, k_ref, v_ref):
    x = x_ref[...]
    rs = lax.rsqrt(jnp.mean(x * x, axis=1, keepdims=True) + EPS)
    xn = x * rs * w1_ref[...]
    qkv = jnp.dot(xn, w_ref[...], preferred_element_type=jnp.float32,
                  precision=_PREC)
    qa = qkv[:, :QD]
    qb = qkv[:, QD:2 * QD]
    ka = qkv[:, 2 * QD:2 * QD + KD]
    kb = qkv[:, 2 * QD + KD:2 * QD + 2 * KD]
    v = qkv[:, 2 * QD + 2 * KD:]
    q = qa * cq_ref[...] + qb * sq_ref[...] + tq_ref[...]
    k = ka * ck_ref[...] + kb * sk_ref[...] + tk_ref[...]
    v = v + vb_ref[...]
    q_ref[...] = (q * SCALE).reshape(BS, H, DH).transpose(1, 0, 2)
    k_ref[...] = k.reshape(BS, KVH, DH).transpose(1, 0, 2)
    v_ref[...] = v.reshape(BS, KVH, DH).transpose(1, 0, 2)


def _run_qkv(x2d, norm1_w, wqkv, cq, sq, ck, sk, tq, tk, v_b):
    full = lambda a: pl.BlockSpec(a.shape, lambda i: (0,) * a.ndim)
    blk = lambda w: pl.BlockSpec((BS, w), lambda i: (i, 0))
    return pl.pallas_call(
        _qkv_body,
        grid=(NB,),
        in_specs=[blk(D), full(norm1_w), full(wqkv),
                  blk(QD), blk(QD), blk(KD), blk(KD),
                  blk(QD), blk(KD), full(v_b)],
        out_specs=[pl.BlockSpec((H, BS, DH), lambda i: (0, i, 0)),
                   pl.BlockSpec((KVH, BS, DH), lambda i: (0, i, 0)),
                   pl.BlockSpec((KVH, BS, DH), lambda i: (0, i, 0))],
        out_shape=[jax.ShapeDtypeStruct((H, S, DH), jnp.float32),
                   jax.ShapeDtypeStruct((KVH, S, DH), jnp.float32),
                   jax.ShapeDtypeStruct((KVH, S, DH), jnp.float32)],
    )(x2d, norm1_w, wqkv, cq, sq, ck, sk, tq, tk, v_b)


def _flash_body(q_ref, k_ref, v_ref, sink_ref, o_ref):
    h = pl.program_id(0)
    qb = pl.program_id(1)
    q = q_ref[0]
    kd = k_ref[0, pl.ds(qb * BQ, BK), :]
    vd = v_ref[0, pl.ds(qb * BQ, BK), :]
    s = lax.dot_general(q, kd, (((1,), (1,)), ((), ())),
                        preferred_element_type=jnp.float32,
                        precision=_PREC)
    rows = lax.broadcasted_iota(jnp.int32, (BQ, BK), 0)
    cols = lax.broadcasted_iota(jnp.int32, (BQ, BK), 1)
    s = jnp.where(cols <= rows, s, -1e30)
    m0 = jnp.max(s, axis=1, keepdims=True)
    p = jnp.exp(s - m0)
    l0 = jnp.sum(p, axis=1, keepdims=True)
    acc0 = lax.dot_general(p, vd, (((1,), (0,)), ((), ())),
                           preferred_element_type=jnp.float32,
                           precision=_PREC)

    def body(kb, carry):
        m, l, acc = carry
        kk = k_ref[0, pl.ds(kb * BK, BK), :]
        vv = v_ref[0, pl.ds(kb * BK, BK), :]
        s = lax.dot_general(q, kk, (((1,), (1,)), ((), ())),
                            preferred_element_type=jnp.float32,
                            precision=_PREC)
        m_new = jnp.maximum(m, jnp.max(s, axis=1, keepdims=True))
        p = jnp.exp(s - m_new)
        alpha = jnp.exp(m - m_new)
        l = l * alpha + jnp.sum(p, axis=1, keepdims=True)
        acc = acc * alpha + lax.dot_general(
            p, vv, (((1,), (0,)), ((), ())),
            preferred_element_type=jnp.float32, precision=_PREC)
        return m_new, l, acc

    m, l, acc = lax.fori_loop(0, qb, body, (m0, l0, acc0))
    sink = sink_ref[0, h]
    m_fin = jnp.maximum(m, sink)
    corr = jnp.exp(m - m_fin)
    l_fin = l * corr + jnp.exp(sink - m_fin)
    o_ref[0] = acc * corr / l_fin


def _run_flash(q, k, v, sinks2d):
    return pl.pallas_call(
        _flash_body,
        grid=(H, NQ),
        in_specs=[
            pl.BlockSpec((1, BQ, DH), lambda h, qb: (h, qb, 0)),
            pl.BlockSpec((1, S, DH), lambda h, qb: (h // (H // KVH), 0, 0)),
            pl.BlockSpec((1, S, DH), lambda h, qb: (h // (H // KVH), 0, 0)),
            pl.BlockSpec(memory_space=pltpu.SMEM),
        ],
        out_specs=pl.BlockSpec((1, BQ, DH), lambda h, qb: (h, qb, 0)),
        out_shape=jax.ShapeDtypeStruct((H, S, DH), jnp.float32),
    )(q, k, v, sinks2d)


def _oproj_body(a_ref, wo_ref, ob_ref, x_ref, w2_ref, wg_ref, gb_ref,
                h_ref, hn_ref, lg_ref):
    attn = a_ref[...].transpose(1, 0, 2).reshape(BS, QD)
    a = jnp.dot(attn, wo_ref[...], preferred_element_type=jnp.float32,
                precision=_PREC)
    hid = a + ob_ref[...] + x_ref[...]
    h_ref[...] = hid
    rs = lax.rsqrt(jnp.mean(hid * hid, axis=1, keepdims=True) + EPS)
    hn = hid * rs * w2_ref[...]
    hn_ref[...] = hn
    lg_ref[...] = jnp.dot(hn, wg_ref[...],
                          preferred_element_type=jnp.float32,
                          precision=_PREC) + gb_ref[...]


def _run_oproj(attn, wo, o_b, x2d, norm2_w, wg_pad, gb_pad):
    full = lambda a: pl.BlockSpec(a.shape, lambda i: (0,) * a.ndim)
    blk = lambda w: pl.BlockSpec((BS, w), lambda i: (i, 0))
    return pl.pallas_call(
        _oproj_body,
        grid=(NB,),
        in_specs=[pl.BlockSpec((H, BS, DH), lambda i: (0, i, 0)),
                  full(wo), full(o_b), blk(D), full(norm2_w),
                  full(wg_pad), full(gb_pad)],
        out_specs=[blk(D), blk(D), blk(128)],
        out_shape=[jax.ShapeDtypeStruct((S, D), jnp.float32),
                   jax.ShapeDtypeStruct((S, D), jnp.float32),
                   jax.ShapeDtypeStruct((S, 128), jnp.float32)],
    )(attn, wo, o_b, x2d, norm2_w, wg_pad, gb_pad)


def _make_sc_gather(n_rows, d, dtype):
    rows_pw = n_rows // SC_NW
    n_chunks = rows_pw // SC_CHUNK
    assert rows_pw % SC_CHUNK == 0
    mesh = plsc.VectorSubcoreMesh(core_axis_name="c", subcore_axis_name="s")

    @functools.partial(
        pl.kernel, mesh=mesh,
        out_type=jax.ShapeDtypeStruct((n_rows, d), dtype),
        scratch_types=[pltpu.VMEM((SC_CHUNK,), jnp.int32),
                       pltpu.VMEM((SC_CHUNK, d), dtype),
                       pltpu.SemaphoreType.DMA],
    )
    def gather(src_hbm, idx_hbm, out_hbm, idx_v, rows_v, sem):
        wid = lax.axis_index("s") * SC_NC + lax.axis_index("c")
        for c in range(n_chunks):
            base = wid * rows_pw + c * SC_CHUNK
            pltpu.sync_copy(idx_hbm.at[pl.ds(base, SC_CHUNK)], idx_v)
            pltpu.async_copy(src_hbm.at[idx_v], rows_v, sem).wait()
            pltpu.sync_copy(rows_v, out_hbm.at[pl.ds(base, SC_CHUNK)])

    return gather


def _ffn_body(be_ref, x_ref, gp_ref, gpb_ref, up_ref, upb_ref,
              dp_ref, dpb_ref, sw_ref, y_ref):
    x = x_ref[...].astype(jnp.bfloat16)
    g = jnp.dot(x, gp_ref[0].astype(jnp.bfloat16),
                preferred_element_type=jnp.float32) + gpb_ref[0]
    u = jnp.dot(x, up_ref[0].astype(jnp.bfloat16),
                preferred_element_type=jnp.float32) + upb_ref[0]
    g = jnp.minimum(g, LIMIT)
    u = jnp.clip(u, -LIMIT, LIMIT)
    glu = g / (1.0 + jnp.exp(-ALPHA * g))
    act = ((u + 1.0) * glu).astype(jnp.bfloat16)
    dn = jnp.dot(act, dp_ref[0].astype(jnp.bfloat16),
                 preferred_element_type=jnp.float32) + dpb_ref[0]
    y_ref[...] = dn * sw_ref[0, 0, :][:, None]


def _run_ffn(x_sorted, block_expert, gp_w, gp_b, up_w, up_b, dp_w, dp_b, sw3):
    grid_spec = pltpu.PrefetchScalarGridSpec(
        num_scalar_prefetch=1,
        grid=(NBLK,),
        in_specs=[
            pl.BlockSpec((BT, D), lambda b, be: (b, 0)),
            pl.BlockSpec((1, D, F), lambda b, be: (be[b], 0, 0)),
            pl.BlockSpec((1, 1, F), lambda b, be: (be[b], 0, 0)),
            pl.BlockSpec((1, D, F), lambda b, be: (be[b], 0, 0)),
            pl.BlockSpec((1, 1, F), lambda b, be: (be[b], 0, 0)),
            pl.BlockSpec((1, F, D), lambda b, be: (be[b], 0, 0)),
            pl.BlockSpec((1, 1, D), lambda b, be: (be[b], 0, 0)),
            pl.BlockSpec((1, 1, BT), lambda b, be: (b, 0, 0)),
        ],
        out_specs=pl.BlockSpec((BT, D), lambda b, be: (b, 0)),
    )
    return pl.pallas_call(
        _ffn_body,
        grid_spec=grid_spec,
        out_shape=jax.ShapeDtypeStruct((NSLOT, D), jnp.float32),
    )(block_expert, x_sorted, gp_w, gp_b, up_w, up_b, dp_w, dp_b, sw3)


def _combine_body(h_ref, y1_ref, y2_ref, o_ref):
    o_ref[...] = h_ref[...] + y1_ref[...] + y2_ref[...]


def _run_combine(h2d, yg):
    blk = pl.BlockSpec((BS, D), lambda i: (i, 0))
    return pl.pallas_call(
        _combine_body,
        grid=(NB,),
        in_specs=[blk,
                  pl.BlockSpec((BS, D), lambda i: (i, 0)),
                  pl.BlockSpec((BS, D), lambda i: (i + NB, 0))],
        out_specs=blk,
        out_shape=jax.ShapeDtypeStruct((S, D), jnp.float32),
    )(h2d, yg, yg)


def _rot_cols(w):
    nh = w.shape[1] // DH
    w3 = w.reshape(w.shape[0], nh, DH)
    return jnp.concatenate([-w3[:, :, DH // 2:], w3[:, :, :DH // 2]],
                           axis=2).reshape(w.shape)


def _rot_vec(b):
    nh = b.shape[0] // DH
    b2 = b.reshape(nh, DH)
    return jnp.concatenate([-b2[:, DH // 2:], b2[:, :DH // 2]],
                           axis=1).reshape(-1)


def kernel(hidden_states, cos, sin, norm1_w, norm2_w, q_w, q_b, k_w, k_b,
           v_w, v_b, o_w, o_b, sinks, gate_w, gate_b, gp_w, gp_b, up_w,
           up_b, dp_w, dp_b):
    x2d = hidden_states.reshape(S, D)
    cos2 = cos.reshape(S, DH)
    sin2 = sin.reshape(S, DH)

    wq = q_w.T
    wk = k_w.T
    wv = v_w.T
    wqkv = jnp.concatenate(
        [wq, _rot_cols(wq), wk, _rot_cols(wk), wv], axis=1)
    cq = jnp.tile(cos2, (1, H))
    sq = jnp.tile(sin2, (1, H))
    ck = jnp.tile(cos2, (1, KVH))
    sk = jnp.tile(sin2, (1, KVH))
    tq = cq * q_b[None, :] + sq * _rot_vec(q_b)[None, :]
    tk = ck * k_b[None, :] + sk * _rot_vec(k_b)[None, :]

    q, k, v = _run_qkv(x2d, norm1_w.reshape(1, D), wqkv, cq, sq, ck, sk,
                       tq, tk, v_b.reshape(1, KD))

    attn = _run_flash(q, k, v, sinks.reshape(1, H))

    wg_pad = jnp.zeros((D, 128), jnp.float32).at[:, :E].set(gate_w.T)
    gb_pad = jnp.zeros((1, 128), jnp.float32).at[0, :E].set(gate_b)
    h2d, hn2d, lg_pad = _run_oproj(
        attn, o_w.T, o_b.reshape(1, D), x2d,
        norm2_w.reshape(1, D), wg_pad, gb_pad)

    logits = lg_pad[:, :E]
    tv, ti = lax.top_k(logits, K)
    tw = jax.nn.softmax(tv, axis=-1)
    expert_ids = ti.reshape(-1)
    pair_w = tw.reshape(-1)
    order = jnp.argsort(expert_ids)
    sorted_e = expert_ids[order]
    counts = jnp.sum(expert_ids[None, :] == jnp.arange(E)[:, None], axis=1)
    padded = ((counts + BT - 1) // BT) * BT
    pad_off = jnp.concatenate([jnp.zeros(1, jnp.int32),
                               jnp.cumsum(padded).astype(jnp.int32)])
    raw_off = jnp.concatenate([jnp.zeros(1, jnp.int32),
                               jnp.cumsum(counts).astype(jnp.int32)])
    slot_pos = pad_off[sorted_e] + (
        jnp.arange(S * K, dtype=jnp.int32) - raw_off[sorted_e])
    slot_token = (jnp.arange(NSLOT, dtype=jnp.int32) % S).at[slot_pos].set(
        (order // K).astype(jnp.int32))
    slot_w = jnp.zeros(NSLOT, jnp.float32).at[slot_pos].set(pair_w[order])
    inv_slot = jnp.zeros(S * K, jnp.int32).at[order].set(slot_pos)
    block_expert = jnp.clip(
        jnp.searchsorted(pad_off, jnp.arange(NBLK, dtype=jnp.int32) * BT,
                         side='right') - 1, 0, E - 1).astype(jnp.int32)

    x_sorted = _make_sc_gather(NSLOT, D, jnp.float32)(hn2d, slot_token)

    y = _run_ffn(x_sorted, block_expert,
                 gp_w, gp_b.reshape(E, 1, F),
                 up_w, up_b.reshape(E, 1, F),
                 dp_w, dp_b.reshape(E, 1, D),
                 slot_w.reshape(NBLK, 1, BT))

    inv2 = inv_slot.reshape(S, K)
    gidx = jnp.concatenate([inv2[:, 0], inv2[:, 1]])
    yg = _make_sc_gather(2 * S, D, jnp.float32)(y, gidx)

    out = _run_combine(h2d, yg)
    return out.reshape(B, S, D)
